# baseline (speedup 1.0000x reference)
import jax
import jax.numpy as jnp
from jax.experimental import pallas as pl
from jax.experimental.pallas import tpu as pltpu

A = 20000
LANES = 128
ROWS = 160
APAD = ROWS * LANES
NUM_OUT = 500
OUT_ROWS = 512
KEPT_ROWS = 4
THRESH = 0.7
NEG_INF = float("-inf")
NB = 2


def _nms_body(scores_in, scoresT_in, anc_ref, del_ref, out_ref,
              box_ref, scT_ref, kept_ref):
    for b in range(NB):
        ay1 = anc_ref[b, 0]
        ax1 = anc_ref[b, 1]
        ay2 = anc_ref[b, 2]
        ax2 = anc_ref[b, 3]
        h = ay2 - ay1
        w = ax2 - ax1
        cy = ay1 + 0.5 * h
        cx = ax1 + 0.5 * w
        cy = cy + del_ref[b, 0] * h
        cx = cx + del_ref[b, 1] * w
        h = h * jnp.exp(del_ref[b, 2])
        w = w * jnp.exp(del_ref[b, 3])
        y1 = jnp.clip(cy - 0.5 * h, 0.0, 1.0)
        x1 = jnp.clip(cx - 0.5 * w, 0.0, 1.0)
        y2 = jnp.clip(cy + 0.5 * h, 0.0, 1.0)
        x2 = jnp.clip(cx + 0.5 * w, 0.0, 1.0)
        box_ref[b, 0] = y1
        box_ref[b, 1] = x1
        box_ref[b, 2] = y2
        box_ref[b, 3] = x2
        box_ref[b, 4] = (y2 - y1) * (x2 - x1)

    scT_ref[...] = scoresT_in[...]
    out_ref[...] = jnp.zeros((NB, OUT_ROWS, 4), jnp.float32)
    kept_ref[...] = jnp.zeros((NB, 5, KEPT_ROWS, LANES), jnp.float32)

    row_iota = jax.lax.broadcasted_iota(jnp.int32, (ROWS, LANES), 0)
    lane_iota = jax.lax.broadcasted_iota(jnp.int32, (1, LANES), 1)
    iota_t = jax.lax.broadcasted_iota(jnp.int32, (1, ROWS), 1)

    cms, cas = [], []
    for b in range(NB):
        sc = scores_in[b]
        cm = jnp.max(sc, axis=0, keepdims=True)
        ca = jnp.min(jnp.where(sc == cm, row_iota, ROWS),
                     axis=0, keepdims=True)
        cms.append(cm)
        cas.append(ca)

    def cond(carry):
        k0, k1, d0, d1 = carry[0], carry[1], carry[2], carry[3]
        return (((k0 < NUM_OUT) & (d0 == 0))
                | ((k1 < NUM_OUT) & (d1 == 0)))

    def body(carry):
        ks = [carry[0], carry[1]]
        ds_ = [carry[2], carry[3]]
        cm = [carry[4], carry[5]]
        ca = [carry[6], carry[7]]
        new_k, new_d, new_cm, new_ca = [], [], [], []
        for b in range(NB):
            active = (ks[b] < NUM_OUT) & (ds_[b] == 0)
            m = jnp.max(cm[b])
            valid = m > NEG_INF
            idx = jnp.min(jnp.where(cm[b] == m,
                                    ca[b] * LANES + lane_iota, APAD))
            r = idx // LANES
            c = idx % LANES
            lm = lane_iota == c
            do_pop = active & valid

            rowT = scT_ref[b, pl.ds(c, 1), :]
            newrow = jnp.where(iota_t == r, NEG_INF, rowT)

            @pl.when(do_pop)
            def _clear():
                scT_ref[b, pl.ds(c, 1), :] = newrow

            cm_c = jnp.max(newrow)
            ca_c = jnp.min(jnp.where(newrow == cm_c, iota_t, ROWS))
            upd = lm & do_pop
            new_cm.append(jnp.where(upd, cm_c, cm[b]))
            new_ca.append(jnp.where(upd, ca_c, ca[b]))

            by1 = jnp.sum(jnp.where(lm, box_ref[b, 0, pl.ds(r, 1), :], 0.0))
            bx1 = jnp.sum(jnp.where(lm, box_ref[b, 1, pl.ds(r, 1), :], 0.0))
            by2 = jnp.sum(jnp.where(lm, box_ref[b, 2, pl.ds(r, 1), :], 0.0))
            bx2 = jnp.sum(jnp.where(lm, box_ref[b, 3, pl.ds(r, 1), :], 0.0))
            yy1 = jnp.maximum(by1, kept_ref[b, 0])
            xx1 = jnp.maximum(bx1, kept_ref[b, 1])
            yy2 = jnp.minimum(by2, kept_ref[b, 2])
            xx2 = jnp.minimum(bx2, kept_ref[b, 3])
            inter = (jnp.maximum(yy2 - yy1, 0.0)
                     * jnp.maximum(xx2 - xx1, 0.0))
            area_b = (by2 - by1) * (bx2 - bx1)
            union = area_b + kept_ref[b, 4] - inter
            iou = inter / jnp.maximum(union, 1e-12)
            keep = do_pop & jnp.logical_not(jnp.any(iou > THRESH))

            krow = ks[b] // LANES
            lm2 = lane_iota == (ks[b] % LANES)

            @pl.when(keep)
            def _append():
                kept_ref[b, 0, pl.ds(krow, 1), :] = jnp.where(
                    lm2, by1, kept_ref[b, 0, pl.ds(krow, 1), :])
                kept_ref[b, 1, pl.ds(krow, 1), :] = jnp.where(
                    lm2, bx1, kept_ref[b, 1, pl.ds(krow, 1), :])
                kept_ref[b, 2, pl.ds(krow, 1), :] = jnp.where(
                    lm2, by2, kept_ref[b, 2, pl.ds(krow, 1), :])
                kept_ref[b, 3, pl.ds(krow, 1), :] = jnp.where(
                    lm2, bx2, kept_ref[b, 3, pl.ds(krow, 1), :])
                kept_ref[b, 4, pl.ds(krow, 1), :] = jnp.where(
                    lm2, area_b, kept_ref[b, 4, pl.ds(krow, 1), :])
                row = jnp.concatenate(
                    [by1.reshape(1, 1), bx1.reshape(1, 1),
                     by2.reshape(1, 1), bx2.reshape(1, 1)], axis=1)
                out_ref[b, pl.ds(ks[b], 1), :] = row

            new_k.append(ks[b] + keep.astype(jnp.int32))
            new_d.append(ds_[b] | (active & ~valid).astype(jnp.int32))

        return (new_k[0], new_k[1], new_d[0], new_d[1],
                new_cm[0], new_cm[1], new_ca[0], new_ca[1])

    jax.lax.while_loop(cond, body, (jnp.int32(0), jnp.int32(0),
                                    jnp.int32(0), jnp.int32(0),
                                    cms[0], cms[1], cas[0], cas[1]))


@jax.jit
def kernel(rpn_probs, bbox_deltas, anchors):
    B = rpn_probs.shape[0]
    pad = APAD - A
    scores = jnp.pad(rpn_probs[:, :, 1], ((0, 0), (0, pad)),
                     constant_values=NEG_INF).reshape(B, ROWS, LANES)
    scoresT = scores.transpose(0, 2, 1)
    anc = jnp.pad(anchors, ((0, 0), (0, pad), (0, 0))).transpose(0, 2, 1)
    anc = anc.reshape(B, 4, ROWS, LANES)
    dlt = jnp.pad(bbox_deltas, ((0, 0), (0, pad), (0, 0))).transpose(0, 2, 1)
    dlt = dlt.reshape(B, 4, ROWS, LANES)

    out = pl.pallas_call(
        _nms_body,
        out_shape=jax.ShapeDtypeStruct((B, OUT_ROWS, 4), jnp.float32),
        scratch_shapes=[
            pltpu.VMEM((NB, 5, ROWS, LANES), jnp.float32),
            pltpu.VMEM((NB, LANES, ROWS), jnp.float32),
            pltpu.VMEM((NB, 5, KEPT_ROWS, LANES), jnp.float32),
        ],
    )(scores, scoresT, anc, dlt)
    return out[:, :NUM_OUT, :]


# speedup vs baseline: 1.4564x; 1.4564x over previous
import jax
import jax.numpy as jnp
from jax.experimental import pallas as pl
from jax.experimental.pallas import tpu as pltpu

A = 20000
LANES = 128
ROWS = 160
APAD = ROWS * LANES
NUM_OUT = 500
OUT_ROWS = 512
KEPT_ROWS = 4
THRESH = 0.7
NEG_INF = float("-inf")
NB = 2
U = 8


def _iou_scalar(a, b):
    yy1 = jnp.maximum(a[0], b[0])
    xx1 = jnp.maximum(a[1], b[1])
    yy2 = jnp.minimum(a[2], b[2])
    xx2 = jnp.minimum(a[3], b[3])
    inter = jnp.maximum(yy2 - yy1, 0.0) * jnp.maximum(xx2 - xx1, 0.0)
    union = a[4] + b[4] - inter
    return inter / jnp.maximum(union, 1e-12)


def _nms_body(scores_in, anc_ref, del_ref, out_ref, box_ref, kept_ref):
    for b in range(NB):
        ay1 = anc_ref[b, 0]
        ax1 = anc_ref[b, 1]
        ay2 = anc_ref[b, 2]
        ax2 = anc_ref[b, 3]
        h = ay2 - ay1
        w = ax2 - ax1
        cy = ay1 + 0.5 * h
        cx = ax1 + 0.5 * w
        cy = cy + del_ref[b, 0] * h
        cx = cx + del_ref[b, 1] * w
        h = h * jnp.exp(del_ref[b, 2])
        w = w * jnp.exp(del_ref[b, 3])
        y1 = jnp.clip(cy - 0.5 * h, 0.0, 1.0)
        x1 = jnp.clip(cx - 0.5 * w, 0.0, 1.0)
        y2 = jnp.clip(cy + 0.5 * h, 0.0, 1.0)
        x2 = jnp.clip(cx + 0.5 * w, 0.0, 1.0)
        box_ref[b, 0] = y1
        box_ref[b, 1] = x1
        box_ref[b, 2] = y2
        box_ref[b, 3] = x2
        box_ref[b, 4] = (y2 - y1) * (x2 - x1)

    out_ref[...] = jnp.zeros((NB, OUT_ROWS, 4), jnp.float32)
    kept_ref[...] = jnp.zeros((NB, 5, KEPT_ROWS, LANES), jnp.float32)

    iota2d = (jax.lax.broadcasted_iota(jnp.int32, (ROWS, LANES), 0) * LANES
              + jax.lax.broadcasted_iota(jnp.int32, (ROWS, LANES), 1))
    lane_iota = jax.lax.broadcasted_iota(jnp.int32, (1, LANES), 1)

    for b in range(NB):
        def cond(carry):
            k, done = carry[0], carry[1]
            return (k < NUM_OUT) & (done == 0)

        def body(carry):
            k, done, scores = carry

            boxes = []
            valids = []
            ious_vs_kept = []
            any_invalid = jnp.int32(0)
            for j in range(U):
                m = jnp.max(scores)
                idx = jnp.min(jnp.where(scores == m, iota2d, APAD))
                valid = m > NEG_INF
                sel = iota2d == idx
                scores = jnp.where(sel, NEG_INF, scores)
                by1 = jnp.sum(jnp.where(sel, box_ref[b, 0], 0.0))
                bx1 = jnp.sum(jnp.where(sel, box_ref[b, 1], 0.0))
                by2 = jnp.sum(jnp.where(sel, box_ref[b, 2], 0.0))
                bx2 = jnp.sum(jnp.where(sel, box_ref[b, 3], 0.0))
                area = (by2 - by1) * (bx2 - bx1)
                boxes.append((by1, bx1, by2, bx2, area))
                valids.append(valid)
                any_invalid = any_invalid | (~valid).astype(jnp.int32)
                yy1 = jnp.maximum(by1, kept_ref[b, 0])
                xx1 = jnp.maximum(bx1, kept_ref[b, 1])
                yy2 = jnp.minimum(by2, kept_ref[b, 2])
                xx2 = jnp.minimum(bx2, kept_ref[b, 3])
                inter = (jnp.maximum(yy2 - yy1, 0.0)
                         * jnp.maximum(xx2 - xx1, 0.0))
                union = area + kept_ref[b, 4] - inter
                iou = inter / jnp.maximum(union, 1e-12)
                ious_vs_kept.append(jnp.any(iou > THRESH))

            keeps = []
            pos = []
            kr = k
            for j in range(U):
                sup = ious_vs_kept[j]
                for i in range(j):
                    sup = sup | (keeps[i] & (_iou_scalar(boxes[i], boxes[j])
                                             > THRESH))
                keep = valids[j] & (kr < NUM_OUT) & jnp.logical_not(sup)
                keeps.append(keep)
                pos.append(kr)
                kr = kr + keep.astype(jnp.int32)

            for j in range(U):
                p = pos[j]
                prow = p // LANES
                lm2 = lane_iota == (p % LANES)
                wmask = keeps[j] & lm2
                by1, bx1, by2, bx2, area = boxes[j]
                for plane, val in ((0, by1), (1, bx1), (2, by2), (3, bx2),
                                   (4, area)):
                    kept_ref[b, plane, pl.ds(prow, 1), :] = jnp.where(
                        wmask, val, kept_ref[b, plane, pl.ds(prow, 1), :])
                row = jnp.concatenate(
                    [by1.reshape(1, 1), bx1.reshape(1, 1),
                     by2.reshape(1, 1), bx2.reshape(1, 1)], axis=1)
                out_ref[b, pl.ds(p, 1), :] = jnp.where(
                    keeps[j], row, out_ref[b, pl.ds(p, 1), :])

            return (kr, done | any_invalid, scores)

        jax.lax.while_loop(cond, body,
                           (jnp.int32(0), jnp.int32(0), scores_in[b]))


@jax.jit
def kernel(rpn_probs, bbox_deltas, anchors):
    B = rpn_probs.shape[0]
    pad = APAD - A
    scores = jnp.pad(rpn_probs[:, :, 1], ((0, 0), (0, pad)),
                     constant_values=NEG_INF).reshape(B, ROWS, LANES)
    anc = jnp.pad(anchors, ((0, 0), (0, pad), (0, 0))).transpose(0, 2, 1)
    anc = anc.reshape(B, 4, ROWS, LANES)
    dlt = jnp.pad(bbox_deltas, ((0, 0), (0, pad), (0, 0))).transpose(0, 2, 1)
    dlt = dlt.reshape(B, 4, ROWS, LANES)

    out = pl.pallas_call(
        _nms_body,
        out_shape=jax.ShapeDtypeStruct((B, OUT_ROWS, 4), jnp.float32),
        scratch_shapes=[
            pltpu.VMEM((NB, 5, ROWS, LANES), jnp.float32),
            pltpu.VMEM((NB, 5, KEPT_ROWS, LANES), jnp.float32),
        ],
    )(scores, anc, dlt)
    return out[:, :NUM_OUT, :]
